# baseline (speedup 1.0000x reference)
import functools

import jax
import jax.numpy as jnp
from jax import lax
from jax.experimental import pallas as pl
from jax.experimental.pallas import tpu as pltpu
from jax.experimental.pallas import tpu_sc as plsc

BATCH = 16384
NUM_FACTORS = 32
NUM_CORES = 2
NUM_SUBCORES = 16
NUM_WORKERS = NUM_CORES * NUM_SUBCORES
ROWS_PER_WORKER = BATCH // NUM_WORKERS
CHUNK = 128
NUM_CHUNKS = ROWS_PER_WORKER // CHUNK
GROUPS = ROWS_PER_WORKER // 16

_mesh = plsc.VectorSubcoreMesh(core_axis_name="c", subcore_axis_name="s")


@functools.partial(
    pl.kernel,
    mesh=_mesh,
    compiler_params=pltpu.CompilerParams(
        needs_layout_passes=False, use_tc_tiling_on_sc=False),
    out_type=jax.ShapeDtypeStruct((BATCH,), jnp.float32),
    scratch_types=[
        pltpu.VMEM((NUM_CHUNKS, CHUNK), jnp.int32),
        pltpu.VMEM((NUM_CHUNKS, CHUNK), jnp.int32),
        pltpu.VMEM((ROWS_PER_WORKER, NUM_FACTORS), jnp.float32),
        pltpu.VMEM((ROWS_PER_WORKER, NUM_FACTORS), jnp.float32),
        pltpu.VMEM((ROWS_PER_WORKER,), jnp.float32),
        pltpu.SemaphoreType.DMA,
    ],
)
def _pmf_sc(ci_hbm, pi_hbm, ct_hbm, pt_hbm, out_hbm,
            ci_v, pi_v, c_rows, p_rows, out_v, sem):
    wid = lax.axis_index("s") * NUM_CORES + lax.axis_index("c")
    base = wid * ROWS_PER_WORKER

    pltpu.sync_copy(ci_hbm.at[pl.ds(wid * NUM_CHUNKS, NUM_CHUNKS)], ci_v)
    pltpu.sync_copy(pi_hbm.at[pl.ds(wid * NUM_CHUNKS, NUM_CHUNKS)], pi_v)

    copies = []
    for k in range(NUM_CHUNKS):
        copies.append(pltpu.async_copy(
            ct_hbm.at[ci_v.at[k]], c_rows.at[pl.ds(k * CHUNK, CHUNK)], sem))
        copies.append(pltpu.async_copy(
            pt_hbm.at[pi_v.at[k]], p_rows.at[pl.ds(k * CHUNK, CHUNK)], sem))
    for c in copies:
        c.wait()

    lanes = lax.iota(jnp.int32, 16)

    def group_body(g, carry):
        base_row = g * 16
        acc = jnp.zeros((16,), jnp.float32)
        for i in range(16):
            r = base_row + i
            c0 = c_rows[r, pl.ds(0, 16)]
            c1 = c_rows[r, pl.ds(16, 16)]
            p0 = p_rows[r, pl.ds(0, 16)]
            p1 = p_rows[r, pl.ds(16, 16)]
            s = jnp.sum(c0 * p0 + c1 * p1)
            acc = jnp.where(lanes == i, s, acc)
        out_v[pl.ds(base_row, 16)] = 1.0 / (1.0 + jnp.exp(-acc))
        return carry

    lax.fori_loop(0, GROUPS, group_body, 0)

    pltpu.sync_copy(out_v, out_hbm.at[pl.ds(base, ROWS_PER_WORKER)])


def kernel(climber_indices, problem_indices, climber_table, problem_table):
    ci = climber_indices.astype(jnp.int32).reshape(NUM_WORKERS * NUM_CHUNKS, CHUNK)
    pi = problem_indices.astype(jnp.int32).reshape(NUM_WORKERS * NUM_CHUNKS, CHUNK)
    return _pmf_sc(ci, pi, climber_table, problem_table)
